# baseline (speedup 1.0000x reference)
import functools
import math

import jax
import jax.numpy as jnp
from jax import lax
from jax.experimental import pallas as pl
from jax.experimental.pallas import tpu as pltpu
from jax.experimental.pallas import tpu_sc as plsc

VOCAB = 32000
PAD = 0
SMOOTH = 0.1
CONF = 1.0 - SMOOTH
EPS = SMOOTH / (VOCAB - 2)
ROW_TLOGT = SMOOTH * math.log(EPS) + CONF * math.log(CONF)

BR = 512
BC = 6400
LANES = 16


def _tc_body(t_ref, x_ref, out_ref):
    r = pl.program_id(0)
    c = pl.program_id(1)

    @pl.when(jnp.logical_and(r == 0, c == 0))
    def _init():
        out_ref[0, 0] = 0.0

    blk = x_ref[...]
    mask = (t_ref[...] != PAD).astype(jnp.float32)
    rowsum = jnp.sum(blk, axis=1, keepdims=True)
    partial = -EPS * jnp.sum(mask * rowsum)

    def first_col_extra():
        return jnp.sum(mask * (ROW_TLOGT + EPS * blk[:, 0:1]))

    partial += jnp.where(c == 0, first_col_extra(), 0.0)
    out_ref[0, 0] += partial


def _tc_part(x, t2d):
    n = x.shape[0]
    grid = (n // BR, VOCAB // BC)
    out = pl.pallas_call(
        _tc_body,
        grid=grid,
        in_specs=[
            pl.BlockSpec((BR, 1), lambda r, c: (r, 0)),
            pl.BlockSpec((BR, BC), lambda r, c: (r, c)),
        ],
        out_specs=pl.BlockSpec(
            (1, 1), lambda r, c: (0, 0), memory_space=pltpu.SMEM),
        out_shape=jax.ShapeDtypeStruct((1, 1), jnp.float32),
        compiler_params=pltpu.CompilerParams(
            dimension_semantics=("arbitrary", "arbitrary")),
    )(t2d, x)
    return out[0, 0]


def _sc_gather(x, tgt, n):
    info = plsc.get_sparse_core_info()
    nc, ns = info.num_cores, info.num_subcores
    nw = nc * ns
    chunk = n // nw
    mesh = plsc.VectorSubcoreMesh(core_axis_name="c", subcore_axis_name="s")

    @functools.partial(
        pl.kernel, mesh=mesh,
        out_type=jax.ShapeDtypeStruct((nw, LANES), jnp.float32),
        scratch_types=[
            pltpu.VMEM((chunk,), jnp.int32),
            pltpu.VMEM((chunk, 128), jnp.float32),
            pltpu.VMEM((LANES,), jnp.float32),
            pltpu.SemaphoreType.DMA,
        ],
    )
    def sc(x_hbm, tgt_hbm, out_g, t_v, c_v, gs_v, sem):
        wid = lax.axis_index("s") * nc + lax.axis_index("c")
        base = wid * chunk
        pltpu.sync_copy(tgt_hbm.at[pl.ds(base, chunk)], t_v)
        ngroups = chunk // LANES

        def fire_group(g, carry):
            off = pl.multiple_of(g * LANES, LANES)
            t16 = t_v[pl.ds(off, LANES)]
            for k in range(LANES):
                t = t16[k]
                col_base = (t // 128) * 128
                pltpu.async_copy(
                    x_hbm.at[base + off + k, pl.ds(col_base, 128)],
                    c_v.at[off + k], sem)
            return carry

        lax.fori_loop(0, ngroups, fire_group, jnp.int32(0))
        pltpu.make_async_copy(
            x_hbm.at[pl.ds(0, chunk), pl.ds(0, 128)], c_v, sem).wait()
        lane_iota = lax.iota(jnp.int32, LANES)

        def extract_group(g, gacc):
            off = pl.multiple_of(g * LANES, LANES)
            t16 = t_v[pl.ds(off, LANES)]
            for k in range(LANES):
                t = t16[k]
                sub_off = pl.multiple_of(((t % 128) // LANES) * LANES, LANES)
                sub = c_v[off + k, pl.ds(sub_off, LANES)]
                sel = jnp.where(lane_iota == (t % LANES), sub, 0.0)
                mval = jnp.where(t != PAD, jnp.float32(1.0), jnp.float32(0.0))
                gacc = gacc + mval * sel
            return gacc

        gs_v[...] = lax.fori_loop(
            0, ngroups, extract_group, jnp.zeros((LANES,), jnp.float32))
        pltpu.sync_copy(gs_v, out_g.at[wid])

    return sc(x, tgt)


def kernel(x, target):
    n = x.shape[0]
    t32 = target.astype(jnp.int32)
    t2d = t32.reshape(n, 1)
    g_part = _sc_gather(x, t32, n)
    tc = _tc_part(x, t2d)
    return (tc - (CONF - EPS) * jnp.sum(g_part)).astype(jnp.float32)


# speedup vs baseline: 1.0218x; 1.0218x over previous
import functools
import math

import jax
import jax.numpy as jnp
from jax import lax
from jax.experimental import pallas as pl
from jax.experimental.pallas import tpu as pltpu
from jax.experimental.pallas import tpu_sc as plsc

VOCAB = 32000
PAD = 0
SMOOTH = 0.1
CONF = 1.0 - SMOOTH
EPS = SMOOTH / (VOCAB - 2)
ROW_TLOGT = SMOOTH * math.log(EPS) + CONF * math.log(CONF)

BR = 512
BC = 6400
LANES = 16


def _tc_body(t_ref, x_ref, out_ref):
    r = pl.program_id(0)
    c = pl.program_id(1)

    @pl.when(jnp.logical_and(r == 0, c == 0))
    def _init():
        out_ref[0, 0] = 0.0

    blk = x_ref[...]
    mask = (t_ref[...] != PAD).astype(jnp.float32)
    rowsum = jnp.sum(blk, axis=1)
    partial = -EPS * jnp.sum(mask * rowsum)

    def first_col_extra():
        return jnp.sum(mask * (ROW_TLOGT + EPS * blk[:, 0]))

    partial += jnp.where(c == 0, first_col_extra(), 0.0)
    out_ref[0, 0] += partial


def _tc_part(x, t1d):
    n = x.shape[0]
    grid = (n // BR, VOCAB // BC)
    out = pl.pallas_call(
        _tc_body,
        grid=grid,
        in_specs=[
            pl.BlockSpec((BR,), lambda r, c: (r,)),
            pl.BlockSpec((BR, BC), lambda r, c: (r, c)),
        ],
        out_specs=pl.BlockSpec(
            (1, 1), lambda r, c: (0, 0), memory_space=pltpu.SMEM),
        out_shape=jax.ShapeDtypeStruct((1, 1), jnp.float32),
        compiler_params=pltpu.CompilerParams(
            dimension_semantics=("arbitrary", "arbitrary")),
    )(t1d, x)
    return out[0, 0]


def _sc_gather(x, tgt, n):
    info = plsc.get_sparse_core_info()
    nc, ns = info.num_cores, info.num_subcores
    nw = nc * ns
    chunk = n // nw
    mesh = plsc.VectorSubcoreMesh(core_axis_name="c", subcore_axis_name="s")

    @functools.partial(
        pl.kernel, mesh=mesh,
        out_type=jax.ShapeDtypeStruct((nw, LANES), jnp.float32),
        scratch_types=[
            pltpu.VMEM((chunk,), jnp.int32),
            pltpu.VMEM((chunk, 128), jnp.float32),
            pltpu.VMEM((LANES,), jnp.float32),
            pltpu.SemaphoreType.DMA,
        ],
    )
    def sc(x_hbm, tgt_hbm, out_g, t_v, c_v, gs_v, sem):
        wid = lax.axis_index("s") * nc + lax.axis_index("c")
        base = wid * chunk
        pltpu.sync_copy(tgt_hbm.at[pl.ds(base, chunk)], t_v)
        ngroups = chunk // LANES

        def fire_group(g, carry):
            off = pl.multiple_of(g * LANES, LANES)
            t16 = t_v[pl.ds(off, LANES)]
            for k in range(LANES):
                t = t16[k]
                col_base = (t // 128) * 128
                pltpu.async_copy(
                    x_hbm.at[base + off + k, pl.ds(col_base, 128)],
                    c_v.at[off + k], sem)
            return carry

        lax.fori_loop(0, ngroups, fire_group, jnp.int32(0))
        pltpu.make_async_copy(
            x_hbm.at[pl.ds(0, chunk), pl.ds(0, 128)], c_v, sem).wait()
        lane_iota = lax.iota(jnp.int32, LANES)

        def extract_group(g, gacc):
            off = pl.multiple_of(g * LANES, LANES)
            t16 = t_v[pl.ds(off, LANES)]
            for k in range(LANES):
                t = t16[k]
                sub_off = pl.multiple_of(((t % 128) // LANES) * LANES, LANES)
                sub = c_v[off + k, pl.ds(sub_off, LANES)]
                sel = jnp.where(lane_iota == (t % LANES), sub, 0.0)
                mval = jnp.where(t != PAD, jnp.float32(1.0), jnp.float32(0.0))
                gacc = gacc + mval * sel
            return gacc

        gs_v[...] = lax.fori_loop(
            0, ngroups, extract_group, jnp.zeros((LANES,), jnp.float32))
        pltpu.sync_copy(gs_v, out_g.at[wid])

    return sc(x, tgt)


def kernel(x, target):
    n = x.shape[0]
    t32 = target.astype(jnp.int32)
    g_part = _sc_gather(x, t32, n)
    tc = _tc_part(x, t32)
    return (tc - (CONF - EPS) * jnp.sum(g_part)).astype(jnp.float32)
